# speedup vs baseline: 1.4405x; 1.4405x over previous
import functools

import jax
import jax.numpy as jnp
from jax import lax
from jax.experimental import pallas as pl
from jax.experimental.pallas import tpu as pltpu
from jax.experimental.pallas import tpu_sc as plsc

_NUM_CORES = 2
_NUM_SUBCORES = 16
_NUM_WORKERS = _NUM_CORES * _NUM_SUBCORES

_CHUNK = 32


@functools.partial(jax.jit, static_argnames=("n_per_w",))
def _embed_gather(idx, table, n_per_w):
    num_rows, d_model = table.shape
    batch = idx.shape[0]
    mesh = plsc.VectorSubcoreMesh(core_axis_name="c", subcore_axis_name="s")

    @functools.partial(
        pl.kernel,
        out_type=jax.ShapeDtypeStruct((batch, d_model), table.dtype),
        mesh=mesh,
        scratch_types=[
            pltpu.VMEM((n_per_w,), jnp.int32),
            pltpu.VMEM((_CHUNK, d_model), table.dtype),
            pltpu.SemaphoreType.DMA,
        ],
    )
    def body(idx_hbm, table_hbm, out_hbm, idx_v, rows_v, sem):
        wid = lax.axis_index("s") * _NUM_CORES + lax.axis_index("c")
        base = wid * n_per_w
        pltpu.sync_copy(idx_hbm.at[pl.ds(base, n_per_w)], idx_v)

        def step(i, carry):
            off = i * _CHUNK
            gather = pltpu.async_copy(
                table_hbm.at[idx_v.at[pl.ds(off, _CHUNK)]], rows_v, sem
            )
            gather.wait()
            pltpu.sync_copy(rows_v, out_hbm.at[pl.ds(base + off, _CHUNK)])
            return carry

        lax.fori_loop(0, n_per_w // _CHUNK, step, 0)

    return body(idx, table)


def kernel(tokens, W_E):
    b, s = tokens.shape
    flat = tokens.reshape(b * s).astype(jnp.int32)
    out = _embed_gather(flat, W_E, (b * s) // _NUM_WORKERS)
    return out.reshape(b, s, W_E.shape[1])


# speedup vs baseline: 1.6948x; 1.1766x over previous
import functools

import jax
import jax.numpy as jnp
from jax import lax
from jax.experimental import pallas as pl
from jax.experimental.pallas import tpu as pltpu
from jax.experimental.pallas import tpu_sc as plsc

_NUM_CORES = 2
_NUM_SUBCORES = 16
_NUM_WORKERS = _NUM_CORES * _NUM_SUBCORES

_CHUNK = 32
_NBUF = 3


@functools.partial(jax.jit, static_argnames=("n_per_w",))
def _embed_gather(idx, table, n_per_w):
    num_rows, d_model = table.shape
    batch = idx.shape[0]
    nch = n_per_w // _CHUNK
    mesh = plsc.VectorSubcoreMesh(core_axis_name="c", subcore_axis_name="s")

    @functools.partial(
        pl.kernel,
        out_type=jax.ShapeDtypeStruct((batch, d_model), table.dtype),
        mesh=mesh,
        scratch_types=[
            pltpu.VMEM((n_per_w,), jnp.int32),
            pltpu.VMEM((_NBUF, _CHUNK, d_model), table.dtype),
            pltpu.SemaphoreType.DMA((_NBUF,)),
            pltpu.SemaphoreType.DMA((_NBUF,)),
        ],
    )
    def body(idx_hbm, table_hbm, out_hbm, idx_v, rows_v, gsem, ssem):
        wid = lax.axis_index("s") * _NUM_CORES + lax.axis_index("c")
        base = wid * n_per_w
        pltpu.sync_copy(idx_hbm.at[pl.ds(base, n_per_w)], idx_v)

        def gather_start(c, b):
            pltpu.async_copy(
                table_hbm.at[idx_v.at[pl.ds(c * _CHUNK, _CHUNK)]],
                rows_v.at[b],
                gsem.at[b],
            )

        def gather_wait(b):
            pltpu.make_async_copy(
                table_hbm.at[idx_v.at[pl.ds(0, _CHUNK)]],
                rows_v.at[b],
                gsem.at[b],
            ).wait()

        def scatter_start(c, b):
            pltpu.async_copy(
                rows_v.at[b],
                out_hbm.at[pl.ds(base + c * _CHUNK, _CHUNK)],
                ssem.at[b],
            )

        def scatter_wait(b):
            pltpu.make_async_copy(
                rows_v.at[b],
                out_hbm.at[pl.ds(base, _CHUNK)],
                ssem.at[b],
            ).wait()

        for b in range(_NBUF - 1):
            gather_start(b, b)

        def step(c, carry):
            b = lax.rem(c, _NBUF)
            nb = lax.rem(c + _NBUF - 1, _NBUF)

            @pl.when(c + _NBUF - 1 < nch)
            def _():
                @pl.when(c >= 1)
                def _():
                    scatter_wait(nb)

                gather_start(c + _NBUF - 1, nb)

            gather_wait(b)
            scatter_start(c, b)
            return carry

        lax.fori_loop(0, nch, step, 0)

        for k in range(_NBUF):
            scatter_wait((nch - _NBUF + k) % _NBUF)

    return body(idx, table)


def kernel(tokens, W_E):
    b, s = tokens.shape
    flat = tokens.reshape(b * s).astype(jnp.int32)
    out = _embed_gather(flat, W_E, (b * s) // _NUM_WORKERS)
    return out.reshape(b, s, W_E.shape[1])


# speedup vs baseline: 1.7041x; 1.0055x over previous
import functools

import jax
import jax.numpy as jnp
from jax import lax
from jax.experimental import pallas as pl
from jax.experimental.pallas import tpu as pltpu
from jax.experimental.pallas import tpu_sc as plsc

_NUM_CORES = 2
_NUM_SUBCORES = 16
_NUM_WORKERS = _NUM_CORES * _NUM_SUBCORES

_CHUNK = 16
_NBUF = 6
_PREF = 3


@functools.partial(jax.jit, static_argnames=("n_per_w",))
def _embed_gather(idx, table, n_per_w):
    num_rows, d_model = table.shape
    batch = idx.shape[0]
    nch = n_per_w // _CHUNK
    mesh = plsc.VectorSubcoreMesh(core_axis_name="c", subcore_axis_name="s")

    @functools.partial(
        pl.kernel,
        out_type=jax.ShapeDtypeStruct((batch, d_model), table.dtype),
        mesh=mesh,
        scratch_types=[
            pltpu.VMEM((n_per_w,), jnp.int32),
            pltpu.VMEM((_NBUF, _CHUNK, d_model), table.dtype),
            pltpu.SemaphoreType.DMA((_NBUF,)),
            pltpu.SemaphoreType.DMA((_NBUF,)),
        ],
    )
    def body(idx_hbm, table_hbm, out_hbm, idx_v, rows_v, gsem, ssem):
        wid = lax.axis_index("s") * _NUM_CORES + lax.axis_index("c")
        base = wid * n_per_w
        pltpu.sync_copy(idx_hbm.at[pl.ds(base, n_per_w)], idx_v)

        def gather_start(c, b):
            pltpu.async_copy(
                table_hbm.at[idx_v.at[pl.ds(c * _CHUNK, _CHUNK)]],
                rows_v.at[b],
                gsem.at[b],
            )

        def gather_wait(b):
            pltpu.make_async_copy(
                table_hbm.at[idx_v.at[pl.ds(0, _CHUNK)]],
                rows_v.at[b],
                gsem.at[b],
            ).wait()

        def scatter_start(c, b):
            pltpu.async_copy(
                rows_v.at[b],
                out_hbm.at[pl.ds(base + c * _CHUNK, _CHUNK)],
                ssem.at[b],
            )

        def scatter_wait(b):
            pltpu.make_async_copy(
                rows_v.at[b],
                out_hbm.at[pl.ds(base, _CHUNK)],
                ssem.at[b],
            ).wait()

        for b in range(_PREF):
            gather_start(b, b)

        def step(c, carry):
            b = lax.rem(c, _NBUF)
            nb = lax.rem(c + _PREF, _NBUF)

            @pl.when(c + _PREF < nch)
            def _():
                @pl.when(c >= _NBUF - _PREF)
                def _():
                    scatter_wait(nb)

                gather_start(c + _PREF, nb)

            gather_wait(b)
            scatter_start(c, b)
            return carry

        lax.fori_loop(0, nch, step, 0)

        for k in range(_NBUF):
            scatter_wait((nch - _NBUF + k) % _NBUF)

    return body(idx, table)


def kernel(tokens, W_E):
    b, s = tokens.shape
    flat = tokens.reshape(b * s).astype(jnp.int32)
    out = _embed_gather(flat, W_E, (b * s) // _NUM_WORKERS)
    return out.reshape(b, s, W_E.shape[1])
